# Initial kernel scaffold; baseline (speedup 1.0000x reference)
import jax
import jax.numpy as jnp
from jax.experimental import pallas as pl


def kernel(x, edge_index, edge_attr, pos, Wq1, Wk1, Wv1, Wq2, Wk2, Wv2):
    raise NotImplementedError("write your pallas kernel here")


# speedup vs baseline: 4.9157x; 4.9157x over previous
import functools

import jax
import jax.numpy as jnp
from jax.experimental import pallas as pl

N = 50000
E = 800000
D_IN = 15
D_EDGE = 5
NUM_RBF = 8
HID = 32
HEADS = 4
HEAD_DIM = HID // HEADS
CUTOFF = 8.0


def _rbf(d):
    centers = jnp.linspace(0.0, CUTOFF, NUM_RBF)
    gamma = CUTOFF / NUM_RBF
    return jnp.exp(-((d[:, None] - centers[None, :]) ** 2) / (gamma ** 2))


def _combine_body(num_ref, den_ref, o_ref, *, relu):
    r = num_ref[...] / (den_ref[...] + 1e-9)
    if relu:
        r = jnp.maximum(r, 0.0)
    o_ref[...] = r


def _combine(num, den, relu):
    den_rep = jnp.repeat(den, HEAD_DIM, axis=1)
    blk = 1000
    return pl.pallas_call(
        functools.partial(_combine_body, relu=relu),
        grid=(N // blk,),
        in_specs=[
            pl.BlockSpec((blk, HID), lambda i: (i, 0)),
            pl.BlockSpec((blk, HID), lambda i: (i, 0)),
        ],
        out_specs=pl.BlockSpec((blk, HID), lambda i: (i, 0)),
        out_shape=jax.ShapeDtypeStruct((N, HID), jnp.float32),
    )(num, den_rep)


def _layer(h, src, dst, eall, Wq, Wk, Wv):
    d_h = h.shape[1]
    n = h.shape[0]
    q = h @ Wq
    kh = h @ Wk[:d_h]
    vh = h @ Wv[:d_h]
    Wke = Wk[d_h:]
    Wve = Wv[d_h:]
    g = jnp.einsum('nhd,jhd->nhj', q.reshape(n, HEADS, HEAD_DIM),
                   Wke.reshape(-1, HEADS, HEAD_DIM))
    qh = q.reshape(n, HEADS, HEAD_DIM)
    khh = kh.reshape(n, HEADS, HEAD_DIM)
    logits = (jnp.einsum('ehd,ehd->eh', qh[dst], khh[src])
              + jnp.einsum('ehj,ej->eh', g[dst], eall)) / jnp.sqrt(float(HEAD_DIM))
    p = jnp.exp(logits)
    ve = (eall @ Wve).reshape(-1, HEADS, HEAD_DIM)
    v = vh[src].reshape(-1, HEADS, HEAD_DIM) + ve
    num = jax.ops.segment_sum((p[..., None] * v).reshape(-1, HID), dst,
                              num_segments=n)
    den = jax.ops.segment_sum(p, dst, num_segments=n)
    return num, den


def kernel(x, edge_index, edge_attr, pos, Wq1, Wk1, Wv1, Wq2, Wk2, Wv2):
    src = edge_index[0]
    dst = edge_index[1]
    diff = pos[dst] - pos[src]
    d = jnp.sqrt((diff * diff).sum(-1) + 1e-9)
    eall = jnp.concatenate([edge_attr, _rbf(d)], axis=1)
    num1, den1 = _layer(x, src, dst, eall, Wq1, Wk1, Wv1)
    h1 = _combine(num1, den1, relu=True)
    num2, den2 = _layer(h1, src, dst, eall, Wq2, Wk2, Wv2)
    h2 = _combine(num2, den2, relu=False)
    return h2


# speedup vs baseline: 23.3338x; 4.7468x over previous
import functools

import jax
import jax.numpy as jnp
from jax import lax
from jax.experimental import pallas as pl
from jax.experimental.pallas import tpu as pltpu
from jax.experimental.pallas import tpu_sc as plsc

N = 50000
E = 800000
D_IN = 15
D_EDGE = 5
NUM_RBF = 8
HID = 32
HEADS = 4
HEAD_DIM = HID // HEADS
CUTOFF = 8.0

NC, NS, L = 2, 16, 16
NW = NC * NS
CHUNK = 64
EP = 802816
NP = 50048
EDGES_PER_TILE = EP // NW
CHUNKS_PER_TILE = EDGES_PER_TILE // CHUNK
ACC_W = HID + HEADS
INV_SQRT_HD = 1.0 / (HEAD_DIM ** 0.5)

_mesh = plsc.VectorSubcoreMesh(core_axis_name="c", subcore_axis_name="s",
                               num_cores=NC, num_subcores=NS)
_sc_params = pltpu.CompilerParams(needs_layout_passes=False,
                                  use_tc_tiling_on_sc=False)

_CENTERS = [CUTOFF * i / (NUM_RBF - 1) for i in range(NUM_RBF)]
_INV_GAMMA = NUM_RBF / CUTOFF


def _rsqrt16(d2):
    ii = plsc.bitcast(d2, jnp.int32)
    ii = 0x5F3759DF - lax.shift_right_arithmetic(ii, 1)
    y = plsc.bitcast(ii, jnp.float32)
    for _ in range(4):
        y = y * (1.5 - 0.5 * d2 * y * y)
    return y


def _full(c):
    return jnp.full((L,), c, jnp.int32)


def _phase0_body(src_hbm, dst_hbm, posp_hbm, eap_hbm, out_hbm,
                 sidx, didx, ps, pd, ea, ob, sem1, sem2, sem3):
    wid = lax.axis_index("s") * NC + lax.axis_index("c")
    tile_base = wid * EDGES_PER_TILE

    def body(i, carry):
        base = tile_base + i * CHUNK
        pltpu.sync_copy(src_hbm.at[pl.ds(base, CHUNK)], sidx)
        pltpu.sync_copy(dst_hbm.at[pl.ds(base, CHUNK)], didx)
        cps = pltpu.async_copy(posp_hbm.at[sidx], ps, sem1)
        cpd = pltpu.async_copy(posp_hbm.at[didx], pd, sem2)
        cea = pltpu.async_copy(eap_hbm.at[pl.ds(base, CHUNK)], ea, sem3)
        cps.wait()
        cpd.wait()
        cea.wait()
        for j in range(CHUNK // L):
            rows = lax.iota(jnp.int32, L) + (j * L)

            def col(ref, c):
                return plsc.load_gather(ref, [rows, _full(c)])

            dx = col(pd, 0) - col(ps, 0)
            dy = col(pd, 1) - col(ps, 1)
            dz = col(pd, 2) - col(ps, 2)
            d2 = dx * dx + dy * dy + dz * dz + 1e-9
            d = d2 * _rsqrt16(d2)
            for r in range(D_EDGE):
                ob[r, pl.ds(j * L, L)] = col(ea, r)
            for r in range(NUM_RBF):
                t = (d - _CENTERS[r]) * _INV_GAMMA
                ob[D_EDGE + r, pl.ds(j * L, L)] = jnp.exp(-(t * t))
            zero = jnp.zeros((L,), jnp.float32)
            for r in range(D_EDGE + NUM_RBF, 16):
                ob[r, pl.ds(j * L, L)] = zero
        pltpu.sync_copy(ob, out_hbm.at[:, pl.ds(base, CHUNK)])
        return carry

    lax.fori_loop(0, CHUNKS_PER_TILE, body, 0)


_phase0 = functools.partial(
    pl.kernel,
    out_type=jax.ShapeDtypeStruct((16, EP), jnp.float32),
    mesh=_mesh,
    scratch_types=[
        pltpu.VMEM((CHUNK,), jnp.int32),
        pltpu.VMEM((CHUNK,), jnp.int32),
        pltpu.VMEM((CHUNK, 16), jnp.float32),
        pltpu.VMEM((CHUNK, 16), jnp.float32),
        pltpu.VMEM((CHUNK, 8), jnp.float32),
        pltpu.VMEM((16, CHUNK), jnp.float32),
        pltpu.SemaphoreType.DMA,
        pltpu.SemaphoreType.DMA,
        pltpu.SemaphoreType.DMA,
    ],
    compiler_params=_sc_params,
)(_phase0_body)


ND = NP // 4


def _layer_body(src_hbm, dst_hbm, q_hbm, s_hbm, ket_hbm, vet_hbm,
                z32_hbm, z16_hbm, opv_hbm, oden_hbm,
                accp, accd, sidx, didx2, didxp, drows, srows, keb, veb,
                pvp, pden, sem1, sem2, sem3, sem4):
    c = lax.axis_index("c")
    s = lax.axis_index("s")
    tile_base = c * (EP // NC) + s * EDGES_PER_TILE
    stripe = pl.ds(s * (NP // NS), NP // NS)
    striped = pl.ds(s * (ND // NS), ND // NS)
    pltpu.sync_copy(z32_hbm.at[stripe], accp.at[stripe])
    pltpu.sync_copy(z16_hbm.at[striped], accd.at[striped])
    plsc.subcore_barrier()
    zero = jnp.zeros((L,), jnp.float32)

    def body(i, carry):
        base = tile_base + i * CHUNK
        pltpu.sync_copy(src_hbm.at[pl.ds(base, CHUNK)], sidx)
        pltpu.sync_copy(dst_hbm.at[pl.ds(base, CHUNK)], didx2.at[0])
        cs = pltpu.async_copy(s_hbm.at[sidx], srows, sem1)
        cd = pltpu.async_copy(q_hbm.at[didx2.at[0]], drows, sem2)
        ck = pltpu.async_copy(ket_hbm.at[:, pl.ds(base, CHUNK)], keb, sem3)
        cv = pltpu.async_copy(vet_hbm.at[:, pl.ds(base, CHUNK)], veb, sem4)
        cs.wait()
        cd.wait()
        ck.wait()
        cv.wait()
        for j in range(CHUNK // L):
            rows = lax.iota(jnp.int32, L) + (j * L)
            dv = didx2[0, pl.ds(j * L, L)]
            didxp[0, pl.ds(j * L, L)] = lax.shift_right_logical(dv, 2)
            dcol = lax.shift_left(jnp.bitwise_and(dv, 3), 2)

            def colq(cc):
                return plsc.load_gather(drows, [rows, _full(cc)])

            def colsrc(cc):
                return plsc.load_gather(srows, [rows, _full(cc)])

            for cc in range(16):
                plsc.store_scatter(pden, [rows, _full(cc)], zero)
            pheads = []
            for h in range(HEADS):
                lg = None
                for d in range(HEAD_DIM):
                    dd = h * HEAD_DIM + d
                    kv = colsrc(dd) + keb[dd, pl.ds(j * L, L)]
                    t = colq(dd) * kv
                    lg = t if lg is None else lg + t
                p_h = jnp.exp(lg * INV_SQRT_HD)
                pheads.append(p_h)
                plsc.store_scatter(pden, [rows, dcol + h], p_h)
            for h in range(HEADS):
                for d in range(HEAD_DIM):
                    dd = h * HEAD_DIM + d
                    v = colsrc(HID + dd) + veb[dd, pl.ds(j * L, L)]
                    plsc.store_scatter(pvp, [rows, _full(dd)], pheads[h] * v)
        pltpu.sync_copy(pvp, accp.at[didx2.at[0]], add=True)
        pltpu.sync_copy(pden, accd.at[didxp.at[0]], add=True)
        return carry

    lax.fori_loop(0, CHUNKS_PER_TILE, body, 0)
    plsc.subcore_barrier()
    pltpu.sync_copy(accp.at[stripe], opv_hbm.at[c, stripe])
    pltpu.sync_copy(accd.at[striped], oden_hbm.at[c, striped])


_layer_sc = functools.partial(
    pl.kernel,
    out_type=[jax.ShapeDtypeStruct((NC, NP, HID), jnp.float32),
              jax.ShapeDtypeStruct((NC, ND, 16), jnp.float32)],
    mesh=_mesh,
    scratch_types=[
        pltpu.VMEM_SHARED((NP, HID), jnp.float32),
        pltpu.VMEM_SHARED((ND, 16), jnp.float32),
        pltpu.VMEM((CHUNK,), jnp.int32),
        pltpu.VMEM((1, CHUNK), jnp.int32),
        pltpu.VMEM((1, CHUNK), jnp.int32),
        pltpu.VMEM((CHUNK, HID), jnp.float32),
        pltpu.VMEM((CHUNK, 2 * HID), jnp.float32),
        pltpu.VMEM((HID, CHUNK), jnp.float32),
        pltpu.VMEM((HID, CHUNK), jnp.float32),
        pltpu.VMEM((CHUNK, HID), jnp.float32),
        pltpu.VMEM((CHUNK, 16), jnp.float32),
        pltpu.SemaphoreType.DMA,
        pltpu.SemaphoreType.DMA,
        pltpu.SemaphoreType.DMA,
        pltpu.SemaphoreType.DMA,
    ],
    compiler_params=_sc_params,
)(_layer_body)


_BC = 28672


def _tca_body(eall_ref, wk1_ref, wv1_ref, wk2_ref, wv2_ref,
              k1_ref, v1_ref, k2_ref, v2_ref):
    eall = eall_ref[...]
    dn = (((0,), (0,)), ((), ()))
    k1_ref[...] = lax.dot_general(wk1_ref[...], eall, dn,
                                  preferred_element_type=jnp.float32)
    v1_ref[...] = lax.dot_general(wv1_ref[...], eall, dn,
                                  preferred_element_type=jnp.float32)
    k2_ref[...] = lax.dot_general(wk2_ref[...], eall, dn,
                                  preferred_element_type=jnp.float32)
    v2_ref[...] = lax.dot_general(wv2_ref[...], eall, dn,
                                  preferred_element_type=jnp.float32)


def _tca(eallT, wke1, wve1, wke2, wve2):
    nb = EP // _BC
    wspec = pl.BlockSpec((16, HID), lambda i: (0, 0))
    ospec = pl.BlockSpec((HID, _BC), lambda i: (0, i))
    return pl.pallas_call(
        _tca_body,
        grid=(nb,),
        in_specs=[pl.BlockSpec((16, _BC), lambda i: (0, i)),
                  wspec, wspec, wspec, wspec],
        out_specs=[ospec, ospec, ospec, ospec],
        out_shape=[jax.ShapeDtypeStruct((HID, EP), jnp.float32)] * 4,
    )(eallT, wke1, wve1, wke2, wve2)


_RB = 6256


def _tcb1_body(x_ref, w_ref, q_ref, s_ref):
    r = jnp.dot(x_ref[...], w_ref[...], preferred_element_type=jnp.float32)
    q_ref[...] = r[:, :HID]
    s_ref[...] = r[:, HID:]


def _tcb1(xp, wcat):
    return pl.pallas_call(
        _tcb1_body,
        grid=(NP // _RB,),
        in_specs=[pl.BlockSpec((_RB, 16), lambda i: (i, 0)),
                  pl.BlockSpec((16, 3 * HID), lambda i: (0, 0))],
        out_specs=[pl.BlockSpec((_RB, HID), lambda i: (i, 0)),
                   pl.BlockSpec((_RB, 2 * HID), lambda i: (i, 0))],
        out_shape=[jax.ShapeDtypeStruct((NP, HID), jnp.float32),
                   jax.ShapeDtypeStruct((NP, 2 * HID), jnp.float32)],
    )(xp, wcat)


def _combine_block(pv, den):
    num = pv[0] + pv[1]
    d = den[0] + den[1]
    rep = jnp.repeat(jnp.eye(HEADS, dtype=jnp.float32), HEAD_DIM, axis=1)
    den_rep = jnp.dot(d, rep, preferred_element_type=jnp.float32)
    return num / (den_rep + 1e-9)


def _tcb2_body(pv_ref, den_ref, w_ref, q_ref, s_ref):
    h1 = jnp.maximum(_combine_block(pv_ref[...], den_ref[...]), 0.0)
    r = jnp.dot(h1, w_ref[...], preferred_element_type=jnp.float32)
    q_ref[...] = r[:, :HID]
    s_ref[...] = r[:, HID:]


def _tcb2(pv, den4, wcat):
    return pl.pallas_call(
        _tcb2_body,
        grid=(NP // _RB,),
        in_specs=[pl.BlockSpec((NC, _RB, HID), lambda i: (0, i, 0)),
                  pl.BlockSpec((NC, _RB, HEADS), lambda i: (0, i, 0)),
                  pl.BlockSpec((HID, 3 * HID), lambda i: (0, 0))],
        out_specs=[pl.BlockSpec((_RB, HID), lambda i: (i, 0)),
                   pl.BlockSpec((_RB, 2 * HID), lambda i: (i, 0))],
        out_shape=[jax.ShapeDtypeStruct((NP, HID), jnp.float32),
                   jax.ShapeDtypeStruct((NP, 2 * HID), jnp.float32)],
    )(pv, den4, wcat)


def _tcc_body(pv_ref, den_ref, o_ref):
    o_ref[...] = _combine_block(pv_ref[...], den_ref[...])


def _tcc(pv, den4):
    return pl.pallas_call(
        _tcc_body,
        grid=(NP // _RB,),
        in_specs=[pl.BlockSpec((NC, _RB, HID), lambda i: (0, i, 0)),
                  pl.BlockSpec((NC, _RB, HEADS), lambda i: (0, i, 0))],
        out_specs=pl.BlockSpec((_RB, HID), lambda i: (i, 0)),
        out_shape=jax.ShapeDtypeStruct((NP, HID), jnp.float32),
    )(pv, den4)


def kernel(x, edge_index, edge_attr, pos, Wq1, Wk1, Wv1, Wq2, Wk2, Wv2):
    src = edge_index[0]
    dst = edge_index[1]
    src_p = jnp.pad(src, (0, EP - E))
    dst_p = jnp.pad(dst, (0, EP - E), constant_values=N)
    posp = jnp.pad(pos, ((0, NP - N), (0, 16 - 3)))
    eap = jnp.pad(edge_attr, ((0, EP - E), (0, 8 - D_EDGE)))

    eallT = _phase0(src_p, dst_p, posp, eap)

    def padw(w):
        return jnp.pad(w, ((0, 16 - w.shape[0]), (0, 0)))

    ket1, vet1, ket2, vet2 = _tca(eallT, padw(Wk1[D_IN:]), padw(Wv1[D_IN:]),
                                  padw(Wk2[HID:]), padw(Wv2[HID:]))

    xp = jnp.pad(x, ((0, NP - N), (0, 16 - D_IN)))
    wcat1 = jnp.pad(jnp.concatenate([Wq1, Wk1[:D_IN], Wv1[:D_IN]], axis=1),
                    ((0, 1), (0, 0)))
    q1, s1 = _tcb1(xp, wcat1)

    z32 = jnp.zeros((NP, HID), jnp.float32)
    z16 = jnp.zeros((ND, 16), jnp.float32)
    pv1, den1 = _layer_sc(src_p, dst_p, q1, s1, ket1, vet1, z32, z16)

    wcat2 = jnp.concatenate([Wq2, Wk2[:HID], Wv2[:HID]], axis=1)
    q2, s2 = _tcb2(pv1, den1.reshape(NC, NP, HEADS), wcat2)

    pv2, den2 = _layer_sc(src_p, dst_p, q2, s2, ket2, vet2, z32, z16)
    h2 = _tcc(pv2, den2.reshape(NC, NP, HEADS))
    return h2[:N]
